# Initial kernel scaffold; baseline (speedup 1.0000x reference)
import jax
import jax.numpy as jnp
from jax.experimental import pallas as pl


def kernel(x, edge_index, batch, W1, b1, W2, b2, Wc1, bc1, Wc2, bc2, Wc3, bc3):
    raise NotImplementedError("write your pallas kernel here")


# speedup vs baseline: 10.9706x; 10.9706x over previous
import functools
import jax
import jax.numpy as jnp
from jax import lax
from jax.experimental import pallas as pl
from jax.experimental.pallas import tpu as pltpu
from jax.experimental.pallas import tpu_sc as plsc

N = 10000
NPAD = 10240
D = 128
E = 320000
G = 64
T = 8
NC = 2
NS = 16
NW = NC * NS
KE = 80
NCH_E = E // (NW * KE)
NCH_E1 = E // (NS * KE)
KN = 80
NCH_N = NPAD // (NW * KN)
DEG_ROWS = 10368
DEG_PER_TILE = DEG_ROWS // NS
DW = 64
THR = 1.0

_f32 = jnp.float32
_mesh = plsc.VectorSubcoreMesh(core_axis_name="c", subcore_axis_name="s")
_mesh1 = plsc.VectorSubcoreMesh(core_axis_name="c", subcore_axis_name="s",
                                num_cores=1)


def _sc_deg_body(dst3, batch3, ones_h, zeros_h, out,
                 ones_v, idx_e, idx_n, zb, acc):
    c = lax.axis_index("c")
    s = lax.axis_index("s")
    wid = c * NS + s
    pltpu.sync_copy(zeros_h, zb)
    pltpu.sync_copy(zb, acc.at[pl.ds(s * DEG_PER_TILE, DEG_PER_TILE)])
    pltpu.sync_copy(ones_h, ones_v)
    pltpu.sync_copy(dst3.at[wid], idx_e)
    pltpu.sync_copy(batch3.at[wid], idx_n)
    plsc.subcore_barrier()

    def step_e(j, carry):
        pltpu.sync_copy(ones_v, acc.at[idx_e.at[j]], add=True)
        return carry

    lax.fori_loop(0, NCH_E, step_e, 0)

    def step_n(j, carry):
        pltpu.sync_copy(ones_v, acc.at[idx_n.at[j]], add=True)
        return carry

    lax.fori_loop(0, NCH_N, step_n, 0)
    plsc.subcore_barrier()
    pltpu.sync_copy(acc.at[pl.ds(s * DEG_PER_TILE, DEG_PER_TILE)], zb)
    pltpu.sync_copy(zb, out.at[pl.ds(c * DEG_ROWS + s * DEG_PER_TILE,
                                     DEG_PER_TILE)])


_sc_deg = pl.kernel(
    _sc_deg_body,
    out_type=jax.ShapeDtypeStruct((NC * DEG_ROWS, DW), _f32),
    mesh=_mesh,
    scratch_types=[
        pltpu.VMEM((KE, DW), _f32),
        pltpu.VMEM((NCH_E, KE), jnp.int32),
        pltpu.VMEM((NCH_N, KN), jnp.int32),
        pltpu.VMEM((DEG_PER_TILE, DW), _f32),
        pltpu.VMEM_SHARED((DEG_ROWS, DW), _f32),
    ],
    compiler_params=pltpu.CompilerParams(use_tc_tiling_on_sc=False),
)


DH = D // 2


def _sc_rowsum_body(tab, src3, dst3, zeros_h, out,
                    src_v, dst_v, rows, zb, acc, gsem):
    c = lax.axis_index("c")
    s = lax.axis_index("s")
    pltpu.sync_copy(zeros_h, zb)
    pltpu.sync_copy(zb, acc.at[pl.ds(s * 640, 640)])
    pltpu.sync_copy(src3.at[s], src_v)
    pltpu.sync_copy(dst3.at[s], dst_v)
    plsc.subcore_barrier()

    def step(j, carry):
        pltpu.async_copy(tab.at[c].at[src_v.at[j]], rows, gsem).wait()
        pltpu.sync_copy(rows, acc.at[dst_v.at[j]], add=True)
        return carry

    lax.fori_loop(0, NCH_E1, step, 0)
    plsc.subcore_barrier()
    pltpu.sync_copy(acc.at[pl.ds(s * 640, 640)], zb)
    pltpu.sync_copy(zb, out.at[pl.ds(c * NPAD + s * 640, 640)])


_sc_rowsum = pl.kernel(
    _sc_rowsum_body,
    out_type=jax.ShapeDtypeStruct((NC * NPAD, DH), _f32),
    mesh=_mesh,
    scratch_types=[
        pltpu.VMEM((NCH_E1, KE), jnp.int32),
        pltpu.VMEM((NCH_E1, KE), jnp.int32),
        pltpu.VMEM((KE, DH), _f32),
        pltpu.VMEM((640, DH), _f32),
        pltpu.VMEM_SHARED((NPAD, DH), _f32),
        pltpu.SemaphoreType.DMA,
    ],
    compiler_params=pltpu.CompilerParams(use_tc_tiling_on_sc=False),
)


def _sc_blur_body(hw, h2, q3, g3, zeros_h, out,
                  q_v, g_v, rows, zb, acc, gsem):
    c = lax.axis_index("c")
    s = lax.axis_index("s")
    wid = c * NS + s
    pltpu.sync_copy(zeros_h, zb)
    pltpu.sync_copy(zb, acc.at[pl.ds(s * 40, 40)])
    pltpu.sync_copy(q3.at[wid], q_v)
    pltpu.sync_copy(g3.at[wid], g_v)
    plsc.subcore_barrier()

    def step(j, carry):
        base = wid * (KN * NCH_N) + j * KN
        pltpu.sync_copy(hw.at[pl.ds(base, KN)], rows)
        pltpu.sync_copy(rows, acc.at[q_v.at[j]], add=True)
        pltpu.sync_copy(h2.at[pl.ds(base, KN)], rows)
        pltpu.sync_copy(rows, acc.at[g_v.at[j]], add=True)
        return carry

    lax.fori_loop(0, NCH_N, step, 0)
    plsc.subcore_barrier()
    pltpu.sync_copy(acc.at[pl.ds(s * 40, 40)], zb)
    pltpu.sync_copy(zb, out.at[c, pl.ds(s * 40, 40)])


_sc_blur = pl.kernel(
    _sc_blur_body,
    out_type=jax.ShapeDtypeStruct((NC, 640, D), _f32),
    mesh=_mesh,
    scratch_types=[
        pltpu.VMEM((NCH_N, KN), jnp.int32),
        pltpu.VMEM((NCH_N, KN), jnp.int32),
        pltpu.VMEM((KN, D), _f32),
        pltpu.VMEM((40, D), _f32),
        pltpu.VMEM_SHARED((640, D), _f32),
        pltpu.SemaphoreType.DMA,
    ],
)


_BLK = 1024
_GRID = NPAD // _BLK


def _dinv_body(degp_ref, out_ref):
    ssum = degp_ref[0] + degp_ref[1]
    out_ref[...] = lax.rsqrt(ssum[:, 0:1] + 1.0)


def _dinv_call(degp):
    return pl.pallas_call(
        _dinv_body,
        grid=(_GRID,),
        in_specs=[pl.BlockSpec((NC, _BLK, DW), lambda i: (0, i, 0))],
        out_specs=pl.BlockSpec((_BLK, 1), lambda i: (i, 0)),
        out_shape=jax.ShapeDtypeStruct((NPAD, 1), _f32),
    )(degp)


def _gmeta_body(cnt_ref, starts_ref, delta_ref, small_ref):
    cnt_col = (cnt_ref[0] + cnt_ref[1])[:, 0:1]
    gi = lax.broadcasted_iota(jnp.int32, (G, G), 0)
    gj = lax.broadcasted_iota(jnp.int32, (G, G), 1)
    cnt_b = jnp.broadcast_to(cnt_col, (G, G))
    counts_row = jnp.sum(jnp.where(gi == gj, cnt_b, 0.0), axis=0,
                         keepdims=True)
    starts_ref[...] = jnp.sum(jnp.where(gi < gj, cnt_b, 0.0), axis=0,
                              keepdims=True)
    r_row = jnp.floor((counts_row + 7.0) / 8.0)
    delta_ref[...] = (0.0 - THR) / jnp.maximum(r_row - 1.0, 1.0)
    small_ref[...] = (cnt_col < 8.0).astype(_f32)


def _gmeta_call(cnt):
    return pl.pallas_call(
        _gmeta_body,
        in_specs=[pl.BlockSpec((NC, G, DW), lambda: (0, 0, 0))],
        out_specs=[
            pl.BlockSpec((1, G), lambda: (0, 0)),
            pl.BlockSpec((1, G), lambda: (0, 0)),
            pl.BlockSpec((G, 1), lambda: (0, 0)),
        ],
        out_shape=[
            jax.ShapeDtypeStruct((1, G), _f32),
            jax.ShapeDtypeStruct((1, G), _f32),
            jax.ShapeDtypeStruct((G, 1), _f32),
        ],
    )(cnt)


def _mm1_body(x_ref, w_ref, dinv_ref, h1_ref, hs1_ref):
    h = jnp.dot(x_ref[...], w_ref[...], preferred_element_type=_f32)
    h1_ref[...] = h
    hs = h * dinv_ref[...]
    hs1_ref[0] = hs[:, :DH]
    hs1_ref[1] = hs[:, DH:]


def _mm1_call(xp, W1, dinv):
    return pl.pallas_call(
        _mm1_body,
        grid=(_GRID,),
        in_specs=[
            pl.BlockSpec((_BLK, D), lambda i: (i, 0)),
            pl.BlockSpec((D, D), lambda i: (0, 0)),
            pl.BlockSpec((_BLK, 1), lambda i: (i, 0)),
        ],
        out_specs=[
            pl.BlockSpec((_BLK, D), lambda i: (i, 0)),
            pl.BlockSpec((NC, _BLK, DH), lambda i: (0, i, 0)),
        ],
        out_shape=[
            jax.ShapeDtypeStruct((NPAD, D), _f32),
            jax.ShapeDtypeStruct((NC, NPAD, DH), _f32),
        ],
    )(xp, W1, dinv)


def _comb1_body(accp_ref, h1_ref, dinv_ref, b1_ref, w2_ref,
                pre_ref, hs2_ref):
    dinv = dinv_ref[...]
    a = jnp.concatenate([accp_ref[0], accp_ref[1]], axis=1)
    h2in = jnp.maximum(a * dinv + dinv * dinv * h1_ref[...] + b1_ref[...],
                       0.0)
    pre = jnp.dot(h2in, w2_ref[...], preferred_element_type=_f32)
    pre_ref[...] = pre
    hs = pre * dinv
    hs2_ref[0] = hs[:, :DH]
    hs2_ref[1] = hs[:, DH:]


def _comb1_call(accp, h1, dinv, b1, W2):
    return pl.pallas_call(
        _comb1_body,
        grid=(_GRID,),
        in_specs=[
            pl.BlockSpec((NC, _BLK, DH), lambda i: (0, i, 0)),
            pl.BlockSpec((_BLK, D), lambda i: (i, 0)),
            pl.BlockSpec((_BLK, 1), lambda i: (i, 0)),
            pl.BlockSpec((1, D), lambda i: (0, 0)),
            pl.BlockSpec((D, D), lambda i: (0, 0)),
        ],
        out_specs=[
            pl.BlockSpec((_BLK, D), lambda i: (i, 0)),
            pl.BlockSpec((NC, _BLK, DH), lambda i: (0, i, 0)),
        ],
        out_shape=[
            jax.ShapeDtypeStruct((NPAD, D), _f32),
            jax.ShapeDtypeStruct((NC, NPAD, DH), _f32),
        ],
    )(accp, h1, dinv, b1, W2)


def _comb2_body(accp_ref, pre_ref, dinv_ref, b2_ref, batch_ref,
                starts_ref, delta_ref,
                h2_ref, hw_ref, q_ref, g_ref):
    i = pl.program_id(0)
    dinv = dinv_ref[...]
    a = jnp.concatenate([accp_ref[0], accp_ref[1]], axis=1)
    h2 = a * dinv + dinv * dinv * pre_ref[...] + b2_ref[...]
    h2_ref[...] = h2
    b = batch_ref[...]
    lane = lax.broadcasted_iota(jnp.int32, (_BLK, G), 1)
    m = (b == lane).astype(_f32)
    starts_i = jnp.sum(m * starts_ref[...], axis=1, keepdims=True)
    delta_i = jnp.sum(m * delta_ref[...], axis=1, keepdims=True)
    i_glob = i * _BLK + lax.broadcasted_iota(jnp.int32, (_BLK, 1), 0)
    p = i_glob.astype(_f32) - starts_i
    cc = jnp.floor(p * 0.125)
    tt = p - 8.0 * cc
    w = THR + cc * delta_i
    hw_ref[...] = w * h2
    ti = tt.astype(jnp.int32)
    q_ref[...] = jnp.where(b >= G, 639, ti * G + b)
    g_ref[...] = jnp.where(b >= G, 639, 512 + b)


def _comb2_call(accp, pre, dinv, b2, batchp, starts, delta):
    return pl.pallas_call(
        _comb2_body,
        grid=(_GRID,),
        in_specs=[
            pl.BlockSpec((NC, _BLK, DH), lambda i: (0, i, 0)),
            pl.BlockSpec((_BLK, D), lambda i: (i, 0)),
            pl.BlockSpec((_BLK, 1), lambda i: (i, 0)),
            pl.BlockSpec((1, D), lambda i: (0, 0)),
            pl.BlockSpec((_BLK, 1), lambda i: (i, 0)),
            pl.BlockSpec((1, G), lambda i: (0, 0)),
            pl.BlockSpec((1, G), lambda i: (0, 0)),
        ],
        out_specs=[
            pl.BlockSpec((_BLK, D), lambda i: (i, 0)),
            pl.BlockSpec((_BLK, D), lambda i: (i, 0)),
            pl.BlockSpec((_BLK, 1), lambda i: (i, 0)),
            pl.BlockSpec((_BLK, 1), lambda i: (i, 0)),
        ],
        out_shape=[
            jax.ShapeDtypeStruct((NPAD, D), _f32),
            jax.ShapeDtypeStruct((NPAD, D), _f32),
            jax.ShapeDtypeStruct((NPAD, 1), jnp.int32),
            jax.ShapeDtypeStruct((NPAD, 1), jnp.int32),
        ],
    )(accp, pre, dinv, b2, batchp, starts, delta)


def _lif(z):
    mem = jnp.zeros_like(z)
    acc = jnp.zeros_like(z)
    for _ in range(4):
        reset = (mem > 1.0).astype(_f32)
        mem = 0.9 * mem + z - reset * 1.0
        acc = acc + (mem > 1.0).astype(_f32)
    return acc * 0.25


def _clf_body(bp_ref, small_ref, wc1_ref, bc1_ref, wc2_ref, bc2_ref,
              wc3_ref, bc3_ref, out_ref):
    big = bp_ref[0] + bp_ref[1]
    gsum = big[512:576]
    small = small_ref[...]
    z1 = jnp.broadcast_to(bc1_ref[...], (G, D)).astype(_f32)
    for t in range(T):
        bt = big[t * G:(t + 1) * G]
        st = (1.0 - t / 7.0) * gsum
        sel = jnp.where(small > 0.0, st, bt)
        z1 = z1 + jnp.dot(sel, wc1_ref[t * D:(t + 1) * D, :],
                          preferred_element_type=_f32)
    z1 = _lif(z1)
    z2 = _lif(jnp.dot(z1, wc2_ref[...], preferred_element_type=_f32)
              + bc2_ref[...])
    out_ref[...] = (jnp.dot(z2, wc3_ref[...], preferred_element_type=_f32)
                    + bc3_ref[...])


def _clf_call(bp, small, Wc1, bc1, Wc2, bc2, Wc3p, bc3p):
    return pl.pallas_call(
        _clf_body,
        in_specs=[
            pl.BlockSpec((NC, 640, D), lambda: (0, 0, 0)),
            pl.BlockSpec((G, 1), lambda: (0, 0)),
            pl.BlockSpec((T * D, D), lambda: (0, 0)),
            pl.BlockSpec((1, D), lambda: (0, 0)),
            pl.BlockSpec((D, D), lambda: (0, 0)),
            pl.BlockSpec((1, D), lambda: (0, 0)),
            pl.BlockSpec((D, D), lambda: (0, 0)),
            pl.BlockSpec((1, D), lambda: (0, 0)),
        ],
        out_specs=pl.BlockSpec((G, D), lambda: (0, 0)),
        out_shape=jax.ShapeDtypeStruct((G, D), _f32),
    )(bp, small, Wc1, bc1, Wc2, bc2, Wc3p, bc3p)


@jax.jit
def _run(x, edge_index, batch, W1, b1, W2, b2, Wc1, bc1, Wc2, bc2, Wc3, bc3):
    src = edge_index[0]
    dst = edge_index[1]

    xp = jnp.pad(x, ((0, NPAD - N), (0, 0)))
    src3 = src.reshape(NS, NCH_E1, KE)
    dst3 = dst.reshape(NS, NCH_E1, KE)
    dst3d = dst.reshape(NW, NCH_E, KE)
    batchp = jnp.pad(batch, (0, NPAD - N), constant_values=G).reshape(NPAD, 1)
    bd = jnp.pad(batch, (0, NPAD - N), constant_values=DEG_ROWS - 1 - NPAD)
    batch3 = (bd + NPAD).reshape(NW, NCH_N, KN)
    ones_h = jnp.ones((KE, DW), _f32)
    zeros_deg = jnp.zeros((DEG_PER_TILE, DW), _f32)
    zeros_rows = jnp.zeros((640, DH), _f32)
    zeros_blur = jnp.zeros((40, D), _f32)
    b1r = b1.reshape(1, D)
    b2r = b2.reshape(1, D)
    bc1r = bc1.reshape(1, D)
    bc2r = bc2.reshape(1, D)
    Wc3p = jnp.pad(Wc3, ((0, 0), (0, D - Wc3.shape[1])))
    bc3p = jnp.pad(bc3, (0, D - bc3.shape[0])).reshape(1, D)

    degout = _sc_deg(dst3d, batch3, ones_h, zeros_deg).reshape(
        NC, DEG_ROWS, DW)
    dinv = _dinv_call(degout[:, :NPAD, :])
    starts, delta, small = _gmeta_call(degout[:, NPAD:NPAD + G, :])
    h1, hs1 = _mm1_call(xp, W1, dinv)
    acc1 = _sc_rowsum(hs1, src3, dst3, zeros_rows).reshape(NC, NPAD, DH)
    pre2, hs2 = _comb1_call(acc1, h1, dinv, b1r, W2)
    acc2 = _sc_rowsum(hs2, src3, dst3, zeros_rows).reshape(NC, NPAD, DH)
    h2, hw, q, gidx = _comb2_call(acc2, pre2, dinv, b2r, batchp,
                                  starts, delta)
    q3 = q.reshape(NW, NCH_N, KN)
    g3 = gidx.reshape(NW, NCH_N, KN)
    blur = _sc_blur(hw, h2, q3, g3, zeros_blur)
    zfull = _clf_call(blur, small, Wc1, bc1r, Wc2, bc2r, Wc3p, bc3p)
    return zfull[:, :Wc3.shape[1]]


def kernel(x, edge_index, batch, W1, b1, W2, b2, Wc1, bc1, Wc2, bc2, Wc3, bc3):
    return _run(x, edge_index, batch, W1, b1, W2, b2,
                Wc1, bc1, Wc2, bc2, Wc3, bc3)


# speedup vs baseline: 17.6323x; 1.6072x over previous
import functools
import jax
import jax.numpy as jnp
from jax import lax
from jax.experimental import pallas as pl
from jax.experimental.pallas import tpu as pltpu
from jax.experimental.pallas import tpu_sc as plsc

N = 10000
NPAD = 10240
D = 128
E = 320000
G = 64
T = 8
NC = 2
NS = 16
NW = NC * NS
KE = 125
NCH_E = E // (NW * KE)
NCH_E1 = E // (NS * KE)
KN = 80
NCH_N = NPAD // (NW * KN)
NHALF = NPAD // 2
DEG_ROWS = 5248
DEG_PER_TILE = DEG_ROWS // NS
DW = 64
THR = 1.0

_f32 = jnp.float32
_mesh = plsc.VectorSubcoreMesh(core_axis_name="c", subcore_axis_name="s")
_mesh1 = plsc.VectorSubcoreMesh(core_axis_name="c", subcore_axis_name="s",
                                num_cores=1)


NCH_N1 = NPAD // (NS * KN)


def _sc_deg_body(dstd, batchd, ones_h, zeros_h, out,
                 ones_v, idx_e, idx_n, zb, acc, dsem, dsem1):
    c = lax.axis_index("c")
    s = lax.axis_index("s")
    pltpu.sync_copy(zeros_h, zb)
    pltpu.sync_copy(zb, acc.at[pl.ds(s * DEG_PER_TILE, DEG_PER_TILE)])
    pltpu.sync_copy(ones_h, ones_v)
    pltpu.sync_copy(dstd.at[c].at[s], idx_e)
    pltpu.sync_copy(batchd.at[c].at[s], idx_n)
    plsc.subcore_barrier()

    dsems = (dsem, dsem1)
    pltpu.async_copy(ones_v, acc.at[idx_e.at[0]], dsems[0], add=True)

    def outer_e(jj, carry):
        for b in range(2):
            j = jj * 2 + b
            pltpu.async_copy(ones_v, acc.at[idx_e.at[j + 1]],
                             dsems[1 - b], add=True)
            pltpu.make_async_copy(ones_v, acc.at[idx_e.at[j]],
                                  dsems[b]).wait()
        return carry

    lax.fori_loop(0, NCH_E1 // 2, outer_e, 0)
    pltpu.make_async_copy(ones_v, acc.at[idx_e.at[NCH_E1]], dsems[0]).wait()

    def step_n(j, carry):
        pltpu.sync_copy(ones_v.at[pl.ds(0, KN)], acc.at[idx_n.at[j]],
                        add=True)
        return carry

    lax.fori_loop(0, NCH_N1, step_n, 0)
    plsc.subcore_barrier()
    pltpu.sync_copy(acc.at[pl.ds(s * DEG_PER_TILE, DEG_PER_TILE)], zb)
    pltpu.sync_copy(zb, out.at[pl.ds(c * DEG_ROWS + s * DEG_PER_TILE,
                                     DEG_PER_TILE)])


_sc_deg = pl.kernel(
    _sc_deg_body,
    out_type=jax.ShapeDtypeStruct((NC * DEG_ROWS, DW), _f32),
    mesh=_mesh,
    scratch_types=[
        pltpu.VMEM((KE, DW), _f32),
        pltpu.VMEM((NCH_E1 + 1, KE), jnp.int32),
        pltpu.VMEM((NCH_N1, KN), jnp.int32),
        pltpu.VMEM((DEG_PER_TILE, DW), _f32),
        pltpu.VMEM_SHARED((DEG_ROWS, DW), _f32),
        pltpu.SemaphoreType.DMA,
        pltpu.SemaphoreType.DMA,
    ],
    compiler_params=pltpu.CompilerParams(use_tc_tiling_on_sc=False),
)


DH = D // 2


def _sc_rowsum_body(tab, src3, dst3, zeros_h, out,
                    src_v, dst_v, rows0, rows1, acc, sem0, sem1):
    c = lax.axis_index("c")
    s = lax.axis_index("s")
    stage = rows0.at[pl.ds(0, 64)]
    pltpu.sync_copy(zeros_h, stage)

    def zstep(k, carry):
        pltpu.sync_copy(stage, acc.at[pl.ds(s * 640 + k * 64, 64)])
        return carry

    lax.fori_loop(0, 10, zstep, 0)
    pltpu.sync_copy(src3.at[s], src_v)
    pltpu.sync_copy(dst3.at[s], dst_v)
    plsc.subcore_barrier()
    tabc = tab.at[c]
    rows = (rows0, rows1)
    sems = (sem0, sem1)
    pltpu.async_copy(tabc.at[src_v.at[0]], rows0, sem0)

    def outer(jj, carry):
        for b in range(2):
            j = jj * 2 + b
            pltpu.async_copy(tabc.at[src_v.at[j + 1]], rows[1 - b],
                             sems[1 - b])
            pltpu.make_async_copy(tabc.at[src_v.at[j]], rows[b],
                                  sems[b]).wait()
            pltpu.sync_copy(rows[b], acc.at[dst_v.at[j]], add=True)
        return carry

    lax.fori_loop(0, NCH_E1 // 2, outer, 0)
    pltpu.make_async_copy(tabc.at[src_v.at[NCH_E1]], rows0, sem0).wait()
    plsc.subcore_barrier()

    def wstep(k, carry):
        pltpu.sync_copy(acc.at[pl.ds(s * 640 + k * 64, 64)], stage)
        pltpu.sync_copy(stage, out.at[pl.ds(c * NPAD + s * 640 + k * 64,
                                            64)])
        return carry

    lax.fori_loop(0, 10, wstep, 0)


_sc_rowsum = pl.kernel(
    _sc_rowsum_body,
    out_type=jax.ShapeDtypeStruct((NC * NPAD, DH), _f32),
    mesh=_mesh,
    scratch_types=[
        pltpu.VMEM((NCH_E1 + 1, KE), jnp.int32),
        pltpu.VMEM((NCH_E1, KE), jnp.int32),
        pltpu.VMEM((KE, DH), _f32),
        pltpu.VMEM((KE, DH), _f32),
        pltpu.VMEM_SHARED((NPAD, DH), _f32),
        pltpu.SemaphoreType.DMA,
        pltpu.SemaphoreType.DMA,
    ],
    compiler_params=pltpu.CompilerParams(use_tc_tiling_on_sc=False),
)


def _sc_blur_body(hw, h2, q3, g3, zeros_h, out,
                  q_v, g_v, rows, zb, acc, gsem):
    c = lax.axis_index("c")
    s = lax.axis_index("s")
    wid = c * NS + s
    pltpu.sync_copy(zeros_h, zb)
    pltpu.sync_copy(zb, acc.at[pl.ds(s * 40, 40)])
    pltpu.sync_copy(q3.at[wid], q_v)
    pltpu.sync_copy(g3.at[wid], g_v)
    plsc.subcore_barrier()

    def step(j, carry):
        base = wid * (KN * NCH_N) + j * KN
        pltpu.sync_copy(hw.at[pl.ds(base, KN)], rows)
        pltpu.sync_copy(rows, acc.at[q_v.at[j]], add=True)
        pltpu.sync_copy(h2.at[pl.ds(base, KN)], rows)
        pltpu.sync_copy(rows, acc.at[g_v.at[j]], add=True)
        return carry

    lax.fori_loop(0, NCH_N, step, 0)
    plsc.subcore_barrier()
    pltpu.sync_copy(acc.at[pl.ds(s * 40, 40)], zb)
    pltpu.sync_copy(zb, out.at[c, pl.ds(s * 40, 40)])


_sc_blur = pl.kernel(
    _sc_blur_body,
    out_type=jax.ShapeDtypeStruct((NC, 640, D), _f32),
    mesh=_mesh,
    scratch_types=[
        pltpu.VMEM((NCH_N, KN), jnp.int32),
        pltpu.VMEM((NCH_N, KN), jnp.int32),
        pltpu.VMEM((KN, D), _f32),
        pltpu.VMEM((40, D), _f32),
        pltpu.VMEM_SHARED((640, D), _f32),
        pltpu.SemaphoreType.DMA,
    ],
)


_BLK = 1024
_GRID = NPAD // _BLK


def _dinv_body(degp_ref, out_ref):
    out_ref[...] = lax.rsqrt(degp_ref[...][:, 0:1] + 1.0)


def _dinv_call(degp):
    return pl.pallas_call(
        _dinv_body,
        grid=(_GRID,),
        in_specs=[pl.BlockSpec((_BLK, DW), lambda i: (i, 0))],
        out_specs=pl.BlockSpec((_BLK, 1), lambda i: (i, 0)),
        out_shape=jax.ShapeDtypeStruct((NPAD, 1), _f32),
    )(degp)


def _gmeta_body(cnt_ref, starts_ref, delta_ref, small_ref):
    cnt_col = cnt_ref[...][:, 0:1]
    gi = lax.broadcasted_iota(jnp.int32, (G, G), 0)
    gj = lax.broadcasted_iota(jnp.int32, (G, G), 1)
    cnt_b = jnp.broadcast_to(cnt_col, (G, G))
    counts_row = jnp.sum(jnp.where(gi == gj, cnt_b, 0.0), axis=0,
                         keepdims=True)
    starts_ref[...] = jnp.sum(jnp.where(gi < gj, cnt_b, 0.0), axis=0,
                              keepdims=True)
    r_row = jnp.floor((counts_row + 7.0) / 8.0)
    delta_ref[...] = (0.0 - THR) / jnp.maximum(r_row - 1.0, 1.0)
    small_ref[...] = (cnt_col < 8.0).astype(_f32)


def _gmeta_call(cnt):
    return pl.pallas_call(
        _gmeta_body,
        in_specs=[pl.BlockSpec((G, DW), lambda: (0, 0))],
        out_specs=[
            pl.BlockSpec((1, G), lambda: (0, 0)),
            pl.BlockSpec((1, G), lambda: (0, 0)),
            pl.BlockSpec((G, 1), lambda: (0, 0)),
        ],
        out_shape=[
            jax.ShapeDtypeStruct((1, G), _f32),
            jax.ShapeDtypeStruct((1, G), _f32),
            jax.ShapeDtypeStruct((G, 1), _f32),
        ],
    )(cnt)


def _mm1_body(x_ref, w_ref, dinv_ref, h1_ref, hs1_ref):
    h = jnp.dot(x_ref[...], w_ref[...], preferred_element_type=_f32)
    h1_ref[...] = h
    hs = h * dinv_ref[...]
    hs1_ref[0] = hs[:, :DH]
    hs1_ref[1] = hs[:, DH:]


def _mm1_call(xp, W1, dinv):
    return pl.pallas_call(
        _mm1_body,
        grid=(_GRID,),
        in_specs=[
            pl.BlockSpec((_BLK, D), lambda i: (i, 0)),
            pl.BlockSpec((D, D), lambda i: (0, 0)),
            pl.BlockSpec((_BLK, 1), lambda i: (i, 0)),
        ],
        out_specs=[
            pl.BlockSpec((_BLK, D), lambda i: (i, 0)),
            pl.BlockSpec((NC, _BLK, DH), lambda i: (0, i, 0)),
        ],
        out_shape=[
            jax.ShapeDtypeStruct((NPAD, D), _f32),
            jax.ShapeDtypeStruct((NC, NPAD, DH), _f32),
        ],
    )(xp, W1, dinv)


def _comb1_body(accp_ref, h1_ref, dinv_ref, b1_ref, w2_ref,
                pre_ref, hs2_ref):
    dinv = dinv_ref[...]
    a = jnp.concatenate([accp_ref[0], accp_ref[1]], axis=1)
    h2in = jnp.maximum(a * dinv + dinv * dinv * h1_ref[...] + b1_ref[...],
                       0.0)
    pre = jnp.dot(h2in, w2_ref[...], preferred_element_type=_f32)
    pre_ref[...] = pre
    hs = pre * dinv
    hs2_ref[0] = hs[:, :DH]
    hs2_ref[1] = hs[:, DH:]


def _comb1_call(accp, h1, dinv, b1, W2):
    return pl.pallas_call(
        _comb1_body,
        grid=(_GRID,),
        in_specs=[
            pl.BlockSpec((NC, _BLK, DH), lambda i: (0, i, 0)),
            pl.BlockSpec((_BLK, D), lambda i: (i, 0)),
            pl.BlockSpec((_BLK, 1), lambda i: (i, 0)),
            pl.BlockSpec((1, D), lambda i: (0, 0)),
            pl.BlockSpec((D, D), lambda i: (0, 0)),
        ],
        out_specs=[
            pl.BlockSpec((_BLK, D), lambda i: (i, 0)),
            pl.BlockSpec((NC, _BLK, DH), lambda i: (0, i, 0)),
        ],
        out_shape=[
            jax.ShapeDtypeStruct((NPAD, D), _f32),
            jax.ShapeDtypeStruct((NC, NPAD, DH), _f32),
        ],
    )(accp, h1, dinv, b1, W2)


def _comb2_body(accp_ref, pre_ref, dinv_ref, b2_ref, batch_ref,
                starts_ref, delta_ref,
                h2_ref, hw_ref, q_ref, g_ref):
    i = pl.program_id(0)
    dinv = dinv_ref[...]
    a = jnp.concatenate([accp_ref[0], accp_ref[1]], axis=1)
    h2 = a * dinv + dinv * dinv * pre_ref[...] + b2_ref[...]
    h2_ref[...] = h2
    b = batch_ref[...]
    lane = lax.broadcasted_iota(jnp.int32, (_BLK, G), 1)
    m = (b == lane).astype(_f32)
    starts_i = jnp.sum(m * starts_ref[...], axis=1, keepdims=True)
    delta_i = jnp.sum(m * delta_ref[...], axis=1, keepdims=True)
    i_glob = i * _BLK + lax.broadcasted_iota(jnp.int32, (_BLK, 1), 0)
    p = i_glob.astype(_f32) - starts_i
    cc = jnp.floor(p * 0.125)
    tt = p - 8.0 * cc
    w = THR + cc * delta_i
    hw_ref[...] = w * h2
    ti = tt.astype(jnp.int32)
    q_ref[...] = jnp.where(b >= G, 639, ti * G + b)
    g_ref[...] = jnp.where(b >= G, 639, 512 + b)


def _comb2_call(accp, pre, dinv, b2, batchp, starts, delta):
    return pl.pallas_call(
        _comb2_body,
        grid=(_GRID,),
        in_specs=[
            pl.BlockSpec((NC, _BLK, DH), lambda i: (0, i, 0)),
            pl.BlockSpec((_BLK, D), lambda i: (i, 0)),
            pl.BlockSpec((_BLK, 1), lambda i: (i, 0)),
            pl.BlockSpec((1, D), lambda i: (0, 0)),
            pl.BlockSpec((_BLK, 1), lambda i: (i, 0)),
            pl.BlockSpec((1, G), lambda i: (0, 0)),
            pl.BlockSpec((1, G), lambda i: (0, 0)),
        ],
        out_specs=[
            pl.BlockSpec((_BLK, D), lambda i: (i, 0)),
            pl.BlockSpec((_BLK, D), lambda i: (i, 0)),
            pl.BlockSpec((_BLK, 1), lambda i: (i, 0)),
            pl.BlockSpec((_BLK, 1), lambda i: (i, 0)),
        ],
        out_shape=[
            jax.ShapeDtypeStruct((NPAD, D), _f32),
            jax.ShapeDtypeStruct((NPAD, D), _f32),
            jax.ShapeDtypeStruct((NPAD, 1), jnp.int32),
            jax.ShapeDtypeStruct((NPAD, 1), jnp.int32),
        ],
    )(accp, pre, dinv, b2, batchp, starts, delta)


def _lif(z):
    mem = jnp.zeros_like(z)
    acc = jnp.zeros_like(z)
    for _ in range(4):
        reset = (mem > 1.0).astype(_f32)
        mem = 0.9 * mem + z - reset * 1.0
        acc = acc + (mem > 1.0).astype(_f32)
    return acc * 0.25


def _clf_body(bp_ref, small_ref, wc1_ref, bc1_ref, wc2_ref, bc2_ref,
              wc3_ref, bc3_ref, out_ref):
    big = bp_ref[0] + bp_ref[1]
    gsum = big[512:576]
    small = small_ref[...]
    z1 = jnp.broadcast_to(bc1_ref[...], (G, D)).astype(_f32)
    for t in range(T):
        bt = big[t * G:(t + 1) * G]
        st = (1.0 - t / 7.0) * gsum
        sel = jnp.where(small > 0.0, st, bt)
        z1 = z1 + jnp.dot(sel, wc1_ref[t * D:(t + 1) * D, :],
                          preferred_element_type=_f32)
    z1 = _lif(z1)
    z2 = _lif(jnp.dot(z1, wc2_ref[...], preferred_element_type=_f32)
              + bc2_ref[...])
    out_ref[...] = (jnp.dot(z2, wc3_ref[...], preferred_element_type=_f32)
                    + bc3_ref[...])


def _clf_call(bp, small, Wc1, bc1, Wc2, bc2, Wc3p, bc3p):
    return pl.pallas_call(
        _clf_body,
        in_specs=[
            pl.BlockSpec((NC, 640, D), lambda: (0, 0, 0)),
            pl.BlockSpec((G, 1), lambda: (0, 0)),
            pl.BlockSpec((T * D, D), lambda: (0, 0)),
            pl.BlockSpec((1, D), lambda: (0, 0)),
            pl.BlockSpec((D, D), lambda: (0, 0)),
            pl.BlockSpec((1, D), lambda: (0, 0)),
            pl.BlockSpec((D, D), lambda: (0, 0)),
            pl.BlockSpec((1, D), lambda: (0, 0)),
        ],
        out_specs=pl.BlockSpec((G, D), lambda: (0, 0)),
        out_shape=jax.ShapeDtypeStruct((G, D), _f32),
    )(bp, small, Wc1, bc1, Wc2, bc2, Wc3p, bc3p)


@jax.jit
def _run(x, edge_index, batch, W1, b1, W2, b2, Wc1, bc1, Wc2, bc2, Wc3, bc3):
    src = edge_index[0]
    dst = edge_index[1]

    xp = jnp.pad(x, ((0, NPAD - N), (0, 0)))
    src2 = src.reshape(NS, NCH_E1, KE)
    src3 = jnp.concatenate([src2, src2[:, :1]], axis=1)
    dst3 = dst.reshape(NS, NCH_E1, KE)
    batchp = jnp.pad(batch, (0, NPAD - N), constant_values=G).reshape(NPAD, 1)
    n_trash = DEG_ROWS - NHALF - G
    tr_e = NHALF + G + (jnp.arange(E, dtype=jnp.int32) % n_trash)
    d0 = jnp.where(dst < NHALF, dst, tr_e)
    d1 = jnp.where(dst >= NHALF, dst - NHALF, tr_e)
    extra = jnp.broadcast_to(
        NHALF + G + (jnp.arange(KE, dtype=jnp.int32) % n_trash),
        (NC, NS, 1, KE)).astype(jnp.int32)
    dstd = jnp.concatenate(
        [jnp.stack([d0, d1]).reshape(NC, NS, NCH_E1, KE), extra], axis=2)
    tr_n = NHALF + G + (jnp.arange(NPAD, dtype=jnp.int32) % n_trash)
    bp_flat = jnp.pad(batch, (0, NPAD - N), constant_values=G)
    bh0 = jnp.where(bp_flat < G, bp_flat + NHALF, tr_n)
    batchd = jnp.stack([bh0, tr_n]).reshape(NC, NS, NCH_N1, KN)
    ones_h = jnp.ones((KE, DW), _f32)
    zeros_deg = jnp.zeros((DEG_PER_TILE, DW), _f32)
    zeros_rows = jnp.zeros((64, DH), _f32)
    zeros_blur = jnp.zeros((40, D), _f32)
    b1r = b1.reshape(1, D)
    b2r = b2.reshape(1, D)
    bc1r = bc1.reshape(1, D)
    bc2r = bc2.reshape(1, D)
    Wc3p = jnp.pad(Wc3, ((0, 0), (0, D - Wc3.shape[1])))
    bc3p = jnp.pad(bc3, (0, D - bc3.shape[0])).reshape(1, D)

    degout = _sc_deg(dstd, batchd, ones_h, zeros_deg)
    degp = jnp.concatenate([degout[:NHALF],
                            degout[DEG_ROWS:DEG_ROWS + NHALF]], axis=0)
    dinv = _dinv_call(degp)
    starts, delta, small = _gmeta_call(degout[NHALF:NHALF + G])
    h1, hs1 = _mm1_call(xp, W1, dinv)
    acc1 = _sc_rowsum(hs1, src3, dst3, zeros_rows).reshape(NC, NPAD, DH)
    pre2, hs2 = _comb1_call(acc1, h1, dinv, b1r, W2)
    acc2 = _sc_rowsum(hs2, src3, dst3, zeros_rows).reshape(NC, NPAD, DH)
    h2, hw, q, gidx = _comb2_call(acc2, pre2, dinv, b2r, batchp,
                                  starts, delta)
    q3 = q.reshape(NW, NCH_N, KN)
    g3 = gidx.reshape(NW, NCH_N, KN)
    blur = _sc_blur(hw, h2, q3, g3, zeros_blur)
    zfull = _clf_call(blur, small, Wc1, bc1r, Wc2, bc2r, Wc3p, bc3p)
    return zfull[:, :Wc3.shape[1]]


def kernel(x, edge_index, batch, W1, b1, W2, b2, Wc1, bc1, Wc2, bc2, Wc3, bc3):
    return _run(x, edge_index, batch, W1, b1, W2, b2,
                Wc1, bc1, Wc2, bc2, Wc3, bc3)
